# baseline (speedup 1.0000x reference)
import functools

import jax
import jax.numpy as jnp
from jax import lax
from jax.experimental import pallas as pl
from jax.experimental.pallas import tpu as pltpu
from jax.experimental.pallas import tpu_sc as plsc


@functools.lru_cache(maxsize=None)
def _make_gather(V, B0, H):
    info = plsc.get_sparse_core_info()
    NC, NS = info.num_cores, info.num_subcores
    NW = NC * NS
    K = 2
    G = 16
    CPG = G // K
    slabs_per_w = B0 // NW
    n_groups = slabs_per_w // G
    assert n_groups >= 2 and (G * H) % 128 == 0
    pieces = []
    off = 0
    while off < H:
        ln = min(128, H - off)
        pieces.append((off, ln))
        off += ln
    mesh = plsc.VectorSubcoreMesh(core_axis_name="c", subcore_axis_name="s")

    @functools.partial(
        pl.kernel,
        mesh=mesh,
        out_type=jax.ShapeDtypeStruct((B0, H, 128), jnp.float32),
        scratch_types=[
            pltpu.VMEM((G * H,), jnp.int32),
            pltpu.VMEM((2, K, H, 128), jnp.float32),
            pltpu.SemaphoreType.DMA,
            pltpu.SemaphoreType.DMA,
        ],
    )
    def gather(table_hbm, idx_hbm, out_hbm, idx_v, rows_v, gsem, ssem):
        wid = lax.axis_index("s") * NC + lax.axis_index("c")
        slab0 = wid * slabs_per_w

        def load_idx(g):
            pltpu.sync_copy(
                idx_hbm.at[pl.ds((slab0 + g * G) * H, G * H)], idx_v
            )

        def fire_gathers(c, j, b):
            for s in range(K):
                for off, ln in pieces:
                    pltpu.async_copy(
                        table_hbm.at[
                            idx_v.at[pl.ds((j * K + s) * H + off, ln)]
                        ],
                        rows_v.at[b].at[s].at[pl.ds(off, ln)],
                        gsem,
                    )

        def wait_gathers(b):
            pltpu.make_async_copy(
                out_hbm.at[pl.ds(0, K)], rows_v.at[b], gsem
            ).wait()

        def fire_store(c, b):
            pltpu.async_copy(
                rows_v.at[b], out_hbm.at[pl.ds(slab0 + c * K, K)], ssem
            )

        def wait_store(b):
            pltpu.make_async_copy(
                rows_v.at[b], out_hbm.at[pl.ds(0, K)], ssem
            ).wait()

        load_idx(0)
        fire_gathers(0, 0, 0)
        fire_gathers(1, 1, 1)
        wait_gathers(0)
        fire_store(0, 0)
        for j in range(2, CPG):
            b = j % 2
            wait_store(b)
            fire_gathers(j, j, b)
            wait_gathers(1 - b)
            fire_store(j - 1, 1 - b)

        def body(g, carry):
            c0 = g * CPG
            wait_gathers(1)
            fire_store(c0 - 1, 1)
            load_idx(g)
            for j in range(CPG):
                b = j % 2
                wait_store(b)
                fire_gathers(c0 + j, j, b)
                if j > 0:
                    wait_gathers(1 - b)
                    fire_store(c0 + j - 1, 1 - b)
            return carry

        lax.fori_loop(1, n_groups, body, 0)

        last = n_groups * CPG - 1
        wait_gathers(1)
        fire_store(last, 1)
        wait_store(0)
        wait_store(1)

    return gather


_BB, _HB = 1024, 8


def _transpose_body_first(i_ref, o_ref):
    D = o_ref.shape[1]
    for h in range(o_ref.shape[0]):
        o_ref[h] = i_ref[:, h, :D].T


def _transpose_body_chain(prev_ref, i_ref, o_ref):
    del prev_ref
    _transpose_body_first(i_ref, o_ref)


@functools.lru_cache(maxsize=None)
def _make_transpose(B0, H, D, B0s, stripe):
    nb = B0s // _BB
    grid = (H // _HB, nb)
    out_spec = pl.BlockSpec(
        (_HB, D, _BB), lambda ih, ib, s=stripe: (ih, 0, ib + s * nb)
    )
    in_spec = pl.BlockSpec((_BB, _HB, 128), lambda ih, ib: (ib, ih, 0))
    out_shape = jax.ShapeDtypeStruct((H, D, B0), jnp.float32)
    if stripe == 0:
        return pl.pallas_call(
            _transpose_body_first,
            grid=grid,
            in_specs=[in_spec],
            out_specs=out_spec,
            out_shape=out_shape,
        )
    return pl.pallas_call(
        _transpose_body_chain,
        grid=grid,
        in_specs=[pl.BlockSpec(memory_space=pl.ANY), in_spec],
        out_specs=out_spec,
        out_shape=out_shape,
        input_output_aliases={0: 0},
    )


def kernel(x, w):
    B0, H = x.shape
    V, D = w.shape
    S = 16
    B0s = B0 // S
    wp = jnp.pad(w, ((0, 0), (0, 128 - D)))
    gs = []
    for s in range(S):
        idx1 = x[s * B0s:(s + 1) * B0s].reshape(B0s * H)
        gs.append(_make_gather(V, B0s, H)(wp, idx1))
    out_t = _make_transpose(B0, H, D, B0s, 0)(gs[0])
    for s in range(1, S):
        out_t = _make_transpose(B0, H, D, B0s, s)(out_t, gs[s])
    return jax.lax.stop_gradient(jnp.transpose(out_t, (2, 0, 1)))


# speedup vs baseline: 1.0091x; 1.0091x over previous
import functools

import jax
import jax.numpy as jnp
from jax import lax
from jax.experimental import pallas as pl
from jax.experimental.pallas import tpu as pltpu
from jax.experimental.pallas import tpu_sc as plsc


@functools.lru_cache(maxsize=None)
def _make_gather(V, B0, H):
    info = plsc.get_sparse_core_info()
    NC, NS = info.num_cores, info.num_subcores
    NW = NC * NS
    K = 2
    G = 16
    CPG = G // K
    slabs_per_w = B0 // NW
    n_groups = slabs_per_w // G
    assert n_groups >= 2 and (G * H) % 128 == 0
    pieces = []
    off = 0
    while off < H:
        ln = min(128, H - off)
        pieces.append((off, ln))
        off += ln
    mesh = plsc.VectorSubcoreMesh(core_axis_name="c", subcore_axis_name="s")

    @functools.partial(
        pl.kernel,
        mesh=mesh,
        out_type=jax.ShapeDtypeStruct((B0, H, 128), jnp.float32),
        scratch_types=[
            pltpu.VMEM((G * H,), jnp.int32),
            pltpu.VMEM((2, K, H, 128), jnp.float32),
            pltpu.SemaphoreType.DMA,
            pltpu.SemaphoreType.DMA,
        ],
    )
    def gather(table_hbm, idx_hbm, out_hbm, idx_v, rows_v, gsem, ssem):
        wid = lax.axis_index("s") * NC + lax.axis_index("c")
        slab0 = wid * slabs_per_w

        def load_idx(g):
            pltpu.sync_copy(
                idx_hbm.at[pl.ds((slab0 + g * G) * H, G * H)], idx_v
            )

        def fire_gathers(c, j, b):
            for s in range(K):
                for off, ln in pieces:
                    pltpu.async_copy(
                        table_hbm.at[
                            idx_v.at[pl.ds((j * K + s) * H + off, ln)]
                        ],
                        rows_v.at[b].at[s].at[pl.ds(off, ln)],
                        gsem,
                    )

        def wait_gathers(b):
            pltpu.make_async_copy(
                out_hbm.at[pl.ds(0, K)], rows_v.at[b], gsem
            ).wait()

        def fire_store(c, b):
            pltpu.async_copy(
                rows_v.at[b], out_hbm.at[pl.ds(slab0 + c * K, K)], ssem
            )

        def wait_store(b):
            pltpu.make_async_copy(
                rows_v.at[b], out_hbm.at[pl.ds(0, K)], ssem
            ).wait()

        load_idx(0)
        fire_gathers(0, 0, 0)
        fire_gathers(1, 1, 1)
        wait_gathers(0)
        fire_store(0, 0)
        for j in range(2, CPG):
            b = j % 2
            wait_store(b)
            fire_gathers(j, j, b)
            wait_gathers(1 - b)
            fire_store(j - 1, 1 - b)

        def body(g, carry):
            c0 = g * CPG
            wait_gathers(1)
            fire_store(c0 - 1, 1)
            load_idx(g)
            for j in range(CPG):
                b = j % 2
                wait_store(b)
                fire_gathers(c0 + j, j, b)
                if j > 0:
                    wait_gathers(1 - b)
                    fire_store(c0 + j - 1, 1 - b)
            return carry

        lax.fori_loop(1, n_groups, body, 0)

        last = n_groups * CPG - 1
        wait_gathers(1)
        fire_store(last, 1)
        wait_store(0)
        wait_store(1)

    return gather


_BB, _HB = 1024, 8


def _transpose_body_first(i_ref, o_ref):
    D = o_ref.shape[1]
    for h in range(o_ref.shape[0]):
        o_ref[h] = i_ref[:, h, :D].T


def _transpose_body_chain(prev_ref, i_ref, o_ref):
    del prev_ref
    _transpose_body_first(i_ref, o_ref)


@functools.lru_cache(maxsize=None)
def _make_transpose(B0, H, D, B0s, stripe):
    nb = B0s // _BB
    grid = (H // _HB, nb)
    out_spec = pl.BlockSpec(
        (_HB, D, _BB), lambda ih, ib, s=stripe: (ih, 0, ib + s * nb)
    )
    in_spec = pl.BlockSpec((_BB, _HB, 128), lambda ih, ib: (ib, ih, 0))
    out_shape = jax.ShapeDtypeStruct((H, D, B0), jnp.float32)
    if stripe == 0:
        return pl.pallas_call(
            _transpose_body_first,
            grid=grid,
            in_specs=[in_spec],
            out_specs=out_spec,
            out_shape=out_shape,
        )
    return pl.pallas_call(
        _transpose_body_chain,
        grid=grid,
        in_specs=[pl.BlockSpec(memory_space=pl.ANY), in_spec],
        out_specs=out_spec,
        out_shape=out_shape,
        input_output_aliases={0: 0},
    )


def kernel(x, w):
    B0, H = x.shape
    V, D = w.shape
    S = 8
    B0s = B0 // S
    wp = jnp.pad(w, ((0, 0), (0, 128 - D)))
    gs = []
    for s in range(S):
        idx1 = x[s * B0s:(s + 1) * B0s].reshape(B0s * H)
        gs.append(_make_gather(V, B0s, H)(wp, idx1))
    out_t = _make_transpose(B0, H, D, B0s, 0)(gs[0])
    for s in range(1, S):
        out_t = _make_transpose(B0, H, D, B0s, s)(out_t, gs[s])
    return jax.lax.stop_gradient(jnp.transpose(out_t, (2, 0, 1)))
